# baseline (speedup 1.0000x reference)
import jax
import jax.numpy as jnp
from jax import lax
from jax.experimental import pallas as pl
from jax.experimental.pallas import tpu as pltpu
from jax.experimental.pallas import tpu_sc as plsc

_NC = 2
_NS = 16
_NW = _NC * _NS
_CH = 128
_EPS = 1e-5


def _mesh():
    return plsc.VectorSubcoreMesh(core_axis_name="c", subcore_axis_name="s")


def _acc_rows(n_nodes):
    return -(-(n_nodes + 1) // (_NS * _CH)) * (_NS * _CH)


def _make_scatter(n_nodes, d, chunks0, chunks1, with_gather):
    sh_rows = _acc_rows(n_nodes)
    zr = sh_rows // _NS // _CH
    cmax = max(chunks0, chunks1)

    scratch = [
        pltpu.VMEM((cmax, _CH), jnp.int32),
        pltpu.VMEM((_CH, d), jnp.float32),
        pltpu.VMEM_SHARED((sh_rows, d), jnp.float32),
        pltpu.SemaphoreType.DMA,
    ]
    if with_gather:
        scratch.insert(0, pltpu.VMEM((cmax, _CH), jnp.int32))

    def body(hbm_refs, out_ref, vmem_refs):
        if with_gather:
            val_hbm, src_hbm, dst_hbm, zeros_hbm = hbm_refs
            src_v, dst_v, rows, s_sh, sem = vmem_refs
        else:
            val_hbm, dst_hbm, zeros_hbm = hbm_refs
            dst_v, rows, s_sh, sem = vmem_refs
        c = lax.axis_index("c")
        s = lax.axis_index("s")
        rowoff = jnp.where(c == 0, s * chunks0, _NS * chunks0 + s * chunks1)
        nch = jnp.where(c == 0, chunks0, chunks1)
        pltpu.sync_copy(dst_hbm.at[pl.ds(rowoff, cmax)], dst_v)
        if with_gather:
            pltpu.sync_copy(src_hbm.at[pl.ds(rowoff, cmax)], src_v)
        pltpu.sync_copy(zeros_hbm, rows)
        for k in range(zr):
            pltpu.sync_copy(rows, s_sh.at[pl.ds((s * zr + k) * _CH, _CH)])
        plsc.subcore_barrier()

        if not with_gather:
            pltpu.sync_copy(val_hbm, rows)

        @pl.loop(0, nch)
        def _(j):
            if with_gather:
                pltpu.async_copy(val_hbm.at[src_v.at[j]], rows, sem).wait()
            pltpu.sync_copy(rows, s_sh.at[dst_v.at[j]], add=True)

        plsc.subcore_barrier()
        for k in range(zr):
            base = (s * zr + k) * _CH
            pltpu.sync_copy(s_sh.at[pl.ds(base, _CH)], rows)
            pltpu.sync_copy(rows, out_ref.at[c, pl.ds(base, _CH)])

    if with_gather:
        def kern(val_hbm, src_hbm, dst_hbm, zeros_hbm, out_ref,
                 src_v, dst_v, rows, s_sh, sem):
            body((val_hbm, src_hbm, dst_hbm, zeros_hbm), out_ref,
                 (src_v, dst_v, rows, s_sh, sem))
    else:
        def kern(val_hbm, dst_hbm, zeros_hbm, out_ref, dst_v, rows, s_sh, sem):
            body((val_hbm, dst_hbm, zeros_hbm), out_ref,
                 (dst_v, rows, s_sh, sem))

    return pl.kernel(
        kern,
        out_type=jax.ShapeDtypeStruct((_NC, sh_rows, d), jnp.float32),
        mesh=_mesh(),
        scratch_types=scratch,
    )


def _t1_body(x_ref, wg_ref, deg_ref, hp_ref, dinv_ref):
    xb = x_ref[...]
    dg = deg_ref[...]
    degsum = dg[0, :, 0:1] + dg[1, :, 0:1] + 1.0
    dinv = lax.rsqrt(jnp.maximum(degsum, 1e-12))
    h = jnp.dot(xb, wg_ref[...], preferred_element_type=jnp.float32)
    dinvb = jnp.broadcast_to(dinv, xb.shape)
    hp_ref[...] = h * dinvb
    dinv_ref[...] = dinvb


def _t1_call(x, wg, deg, bt):
    n, d = x.shape
    grid = n // bt
    return pl.pallas_call(
        _t1_body,
        grid=(grid,),
        in_specs=[
            pl.BlockSpec((bt, d), lambda i: (i, 0)),
            pl.BlockSpec((d, d), lambda i: (0, 0)),
            pl.BlockSpec((_NC, bt, d), lambda i: (0, i, 0)),
        ],
        out_specs=[
            pl.BlockSpec((bt, d), lambda i: (i, 0)),
            pl.BlockSpec((bt, d), lambda i: (i, 0)),
        ],
        out_shape=[
            jax.ShapeDtypeStruct((n, d), jnp.float32),
            jax.ShapeDtypeStruct((n, d), jnp.float32),
        ],
    )(x, wg, deg)


def _ln(v, g, b):
    m = jnp.mean(v, axis=-1, keepdims=True)
    var = jnp.mean((v - m) ** 2, axis=-1, keepdims=True)
    return (v - m) * lax.rsqrt(var + _EPS) * g + b


def _t2_body(x_ref, hp_ref, dinv_ref, s_ref, bg_ref, g1_ref, b1_ref,
             w1_ref, c1_ref, w2_ref, c2_ref, g2_ref, b2_ref, out_ref):
    xb = x_ref[...]
    conv = dinv_ref[...] * (s_ref[0] + s_ref[1] + hp_ref[...]) + bg_ref[...]
    x1 = _ln(xb + conv, g1_ref[...], b1_ref[...])
    h = jnp.maximum(jnp.dot(x1, w1_ref[...], preferred_element_type=jnp.float32)
                    + c1_ref[...], 0.0)
    ffn = jnp.dot(h, w2_ref[...], preferred_element_type=jnp.float32) + c2_ref[...]
    out_ref[...] = _ln(x1 + ffn, g2_ref[...], b2_ref[...])


def _t2_call(x, hp, dinvb, s_part, p, bt):
    n, d = x.shape
    ff = p['W1'].shape[1]
    grid = n // bt
    row = lambda i: (i, 0)
    zero = lambda i: (0, 0)
    vec = lambda a: a.reshape(1, -1)
    return pl.pallas_call(
        _t2_body,
        grid=(grid,),
        in_specs=[
            pl.BlockSpec((bt, d), row),
            pl.BlockSpec((bt, d), row),
            pl.BlockSpec((bt, d), row),
            pl.BlockSpec((_NC, bt, d), lambda i: (0, i, 0)),
            pl.BlockSpec((1, d), zero),
            pl.BlockSpec((1, d), zero),
            pl.BlockSpec((1, d), zero),
            pl.BlockSpec((d, ff), zero),
            pl.BlockSpec((1, ff), zero),
            pl.BlockSpec((ff, d), zero),
            pl.BlockSpec((1, d), zero),
            pl.BlockSpec((1, d), zero),
            pl.BlockSpec((1, d), zero),
        ],
        out_specs=pl.BlockSpec((bt, d), row),
        out_shape=jax.ShapeDtypeStruct((n, d), jnp.float32),
    )(x, hp, dinvb, s_part, vec(p['bg']), vec(p['g1']), vec(p['b1']),
      p['W1'], vec(p['c1']), p['W2'], vec(p['c2']), vec(p['g2']), vec(p['b2']))


def kernel(x, edge_index, params):
    n, d = x.shape
    e = edge_index.shape[1]
    bt = 1000

    pair = -(-e // (_NS * _CH * 16)) * 16
    chunks0 = int(pair * 0.35) // 8 * 8
    chunks1 = pair - chunks0
    cmax = max(chunks0, chunks1)
    rows_used = _NS * pair
    rows_pad = rows_used + cmax
    e_pad = rows_pad * _CH
    src = edge_index[0]
    dst = edge_index[1]
    pad = e_pad - e
    srcp = jnp.concatenate(
        [src, jnp.zeros((pad,), jnp.int32)]).reshape(rows_pad, _CH)
    dstp = jnp.concatenate(
        [dst, jnp.full((pad,), n, jnp.int32)]).reshape(rows_pad, _CH)

    zerosd = jnp.zeros((_CH, d), jnp.float32)
    onesd = jnp.ones((_CH, d), jnp.float32)

    deg = _make_scatter(n, d, chunks0, chunks1, with_gather=False)(
        onesd, dstp, zerosd)
    scat = _make_scatter(n, d, chunks0, chunks1, with_gather=True)

    for p in params:
        hp, dinvb = _t1_call(x, p['Wg'], deg, bt)
        s_part = scat(hp, srcp, dstp, zerosd)
        x = _t2_call(x, hp, dinvb, s_part, p, bt)
    return x


# speedup vs baseline: 1.1473x; 1.1473x over previous
import jax
import jax.numpy as jnp
from jax import lax
from jax.experimental import pallas as pl
from jax.experimental.pallas import tpu as pltpu
from jax.experimental.pallas import tpu_sc as plsc

_NC = 2
_NS = 16
_NW = _NC * _NS
_CH = 128
_EPS = 1e-5


def _mesh():
    return plsc.VectorSubcoreMesh(core_axis_name="c", subcore_axis_name="s")


def _acc_rows(n_nodes):
    return -(-(n_nodes + 1) // (_NS * _CH)) * (_NS * _CH)


def _make_scatter(n_nodes, d, chunks0, chunks1, with_gather):
    sh_rows = _acc_rows(n_nodes)
    zr = sh_rows // _NS // _CH
    cmax = max(chunks0, chunks1)

    scratch = [
        pltpu.VMEM((cmax, _CH), jnp.int32),
        pltpu.VMEM((_CH, d), jnp.float32),
        pltpu.VMEM_SHARED((sh_rows, d), jnp.float32),
        pltpu.SemaphoreType.DMA,
    ]
    if with_gather:
        scratch.insert(0, pltpu.VMEM((cmax, _CH), jnp.int32))

    def body(hbm_refs, out_ref, vmem_refs):
        if with_gather:
            val_hbm, src_hbm, dst_hbm, zeros_hbm = hbm_refs
            src_v, dst_v, rows, s_sh, sem = vmem_refs
        else:
            val_hbm, dst_hbm, zeros_hbm = hbm_refs
            dst_v, rows, s_sh, sem = vmem_refs
        c = lax.axis_index("c")
        s = lax.axis_index("s")
        rowoff = jnp.where(c == 0, s * chunks0, _NS * chunks0 + s * chunks1)
        nch = jnp.where(c == 0, chunks0, chunks1)
        pltpu.sync_copy(dst_hbm.at[pl.ds(rowoff, cmax)], dst_v)
        if with_gather:
            pltpu.sync_copy(src_hbm.at[pl.ds(rowoff, cmax)], src_v)
        pltpu.sync_copy(zeros_hbm, rows)
        for k in range(zr):
            pltpu.sync_copy(rows, s_sh.at[pl.ds((s * zr + k) * _CH, _CH)])
        plsc.subcore_barrier()

        if not with_gather:
            pltpu.sync_copy(val_hbm, rows)

        @pl.loop(0, nch)
        def _(j):
            if with_gather:
                pltpu.async_copy(val_hbm.at[src_v.at[j]], rows, sem).wait()
            pltpu.sync_copy(rows, s_sh.at[dst_v.at[j]], add=True)

        plsc.subcore_barrier()
        for k in range(zr):
            base = (s * zr + k) * _CH
            pltpu.sync_copy(s_sh.at[pl.ds(base, _CH)], rows)
            pltpu.sync_copy(rows, out_ref.at[c, pl.ds(base, _CH)])

    if with_gather:
        def kern(val_hbm, src_hbm, dst_hbm, zeros_hbm, out_ref,
                 src_v, dst_v, rows, s_sh, sem):
            body((val_hbm, src_hbm, dst_hbm, zeros_hbm), out_ref,
                 (src_v, dst_v, rows, s_sh, sem))
    else:
        def kern(val_hbm, dst_hbm, zeros_hbm, out_ref, dst_v, rows, s_sh, sem):
            body((val_hbm, dst_hbm, zeros_hbm), out_ref,
                 (dst_v, rows, s_sh, sem))

    return pl.kernel(
        kern,
        out_type=jax.ShapeDtypeStruct((_NC, sh_rows, d), jnp.float32),
        mesh=_mesh(),
        scratch_types=scratch,
    )


def _t1_body(x_ref, wg_ref, deg_ref, hp_ref, dinv_ref):
    xb = x_ref[...]
    dg = deg_ref[...]
    degsum = dg[0, :, 0:1] + dg[1, :, 0:1] + 1.0
    dinv = lax.rsqrt(jnp.maximum(degsum, 1e-12))
    h = jnp.dot(xb, wg_ref[...], preferred_element_type=jnp.float32)
    dinvb = jnp.broadcast_to(dinv, xb.shape)
    hp_ref[...] = h * dinvb
    dinv_ref[...] = dinvb


def _t1_call(x, wg, deg, bt):
    n, d = x.shape
    grid = n // bt
    return pl.pallas_call(
        _t1_body,
        grid=(grid,),
        in_specs=[
            pl.BlockSpec((bt, d), lambda i: (i, 0)),
            pl.BlockSpec((d, d), lambda i: (0, 0)),
            pl.BlockSpec((_NC, bt, d), lambda i: (0, i, 0)),
        ],
        out_specs=[
            pl.BlockSpec((bt, d), lambda i: (i, 0)),
            pl.BlockSpec((bt, d), lambda i: (i, 0)),
        ],
        out_shape=[
            jax.ShapeDtypeStruct((n, d), jnp.float32),
            jax.ShapeDtypeStruct((n, d), jnp.float32),
        ],
    )(x, wg, deg)


def _ln(v, g, b):
    m = jnp.mean(v, axis=-1, keepdims=True)
    var = jnp.mean((v - m) ** 2, axis=-1, keepdims=True)
    return (v - m) * lax.rsqrt(var + _EPS) * g + b


def _t2_body(x_ref, hp_ref, dinv_ref, s_ref, bg_ref, g1_ref, b1_ref,
             w1_ref, c1_ref, w2_ref, c2_ref, g2_ref, b2_ref, out_ref):
    xb = x_ref[...]
    conv = dinv_ref[...] * (s_ref[0] + s_ref[1] + hp_ref[...]) + bg_ref[...]
    x1 = _ln(xb + conv, g1_ref[...], b1_ref[...])
    h = jnp.maximum(jnp.dot(x1, w1_ref[...], preferred_element_type=jnp.float32)
                    + c1_ref[...], 0.0)
    ffn = jnp.dot(h, w2_ref[...], preferred_element_type=jnp.float32) + c2_ref[...]
    out_ref[...] = _ln(x1 + ffn, g2_ref[...], b2_ref[...])


def _t2_call(x, hp, dinvb, s_part, p, bt):
    n, d = x.shape
    ff = p['W1'].shape[1]
    grid = n // bt
    row = lambda i: (i, 0)
    zero = lambda i: (0, 0)
    vec = lambda a: a.reshape(1, -1)
    return pl.pallas_call(
        _t2_body,
        grid=(grid,),
        in_specs=[
            pl.BlockSpec((bt, d), row),
            pl.BlockSpec((bt, d), row),
            pl.BlockSpec((bt, d), row),
            pl.BlockSpec((_NC, bt, d), lambda i: (0, i, 0)),
            pl.BlockSpec((1, d), zero),
            pl.BlockSpec((1, d), zero),
            pl.BlockSpec((1, d), zero),
            pl.BlockSpec((d, ff), zero),
            pl.BlockSpec((1, ff), zero),
            pl.BlockSpec((ff, d), zero),
            pl.BlockSpec((1, d), zero),
            pl.BlockSpec((1, d), zero),
            pl.BlockSpec((1, d), zero),
        ],
        out_specs=pl.BlockSpec((bt, d), row),
        out_shape=jax.ShapeDtypeStruct((n, d), jnp.float32),
    )(x, hp, dinvb, s_part, vec(p['bg']), vec(p['g1']), vec(p['b1']),
      p['W1'], vec(p['c1']), p['W2'], vec(p['c2']), vec(p['g2']), vec(p['b2']))


def kernel(x, edge_index, params):
    n, d = x.shape
    e = edge_index.shape[1]
    bt = 1000

    pair = -(-e // (_NS * _CH * 16)) * 16
    chunks0 = -(-int(pair * 0.65) // 8) * 8
    chunks1 = pair - chunks0
    cmax = max(chunks0, chunks1)
    rows_used = _NS * pair
    rows_pad = rows_used + cmax
    e_pad = rows_pad * _CH
    src = edge_index[0]
    dst = edge_index[1]
    pad = e_pad - e
    srcp = jnp.concatenate(
        [src, jnp.zeros((pad,), jnp.int32)]).reshape(rows_pad, _CH)
    dstp = jnp.concatenate(
        [dst, jnp.full((pad,), n, jnp.int32)]).reshape(rows_pad, _CH)

    zerosd = jnp.zeros((_CH, d), jnp.float32)
    onesd = jnp.ones((_CH, d), jnp.float32)

    deg = _make_scatter(n, d, chunks0, chunks1, with_gather=False)(
        onesd, dstp, zerosd)
    scat = _make_scatter(n, d, chunks0, chunks1, with_gather=True)

    for p in params:
        hp, dinvb = _t1_call(x, p['Wg'], deg, bt)
        s_part = scat(hp, srcp, dstp, zerosd)
        x = _t2_call(x, hp, dinvb, s_part, p, bt)
    return x


# speedup vs baseline: 1.1702x; 1.0199x over previous
import jax
import jax.numpy as jnp
from jax import lax
from jax.experimental import pallas as pl
from jax.experimental.pallas import tpu as pltpu
from jax.experimental.pallas import tpu_sc as plsc

_NC = 2
_NS = 16
_NW = _NC * _NS
_CH = 128
_EPS = 1e-5


def _mesh():
    return plsc.VectorSubcoreMesh(core_axis_name="c", subcore_axis_name="s")


def _acc_rows(n_nodes):
    return -(-(n_nodes + 1) // (_NS * _CH)) * (_NS * _CH)


def _make_scatter(n_nodes, d, chunks0, chunks1, with_gather):
    sh_rows = _acc_rows(n_nodes)
    zr = sh_rows // _NS // _CH
    cmax = max(chunks0, chunks1)

    scratch = [
        pltpu.VMEM((cmax, _CH), jnp.int32),
        pltpu.VMEM((_CH, d), jnp.float32),
        pltpu.VMEM_SHARED((sh_rows, d), jnp.float32),
        pltpu.SemaphoreType.DMA,
    ]
    if with_gather:
        scratch.insert(0, pltpu.VMEM((cmax, _CH), jnp.int32))

    def body(hbm_refs, out_ref, vmem_refs):
        if with_gather:
            val_hbm, src_hbm, dst_hbm, zeros_hbm = hbm_refs
            src_v, dst_v, rows, s_sh, sem = vmem_refs
        else:
            val_hbm, dst_hbm, zeros_hbm = hbm_refs
            dst_v, rows, s_sh, sem = vmem_refs
        c = lax.axis_index("c")
        s = lax.axis_index("s")
        pltpu.sync_copy(zeros_hbm, rows)
        for k in range(zr):
            pltpu.sync_copy(rows, s_sh.at[pl.ds((s * zr + k) * _CH, _CH)])
        plsc.subcore_barrier()

        if not with_gather:
            pltpu.sync_copy(val_hbm, rows)

        for cc, nch, base in ((0, chunks0, 0), (1, chunks1, _NS * chunks0)):
            @pl.when(c == cc)
            def _():
                rowoff = base + s * nch
                pltpu.sync_copy(dst_hbm.at[pl.ds(rowoff, nch)],
                                dst_v.at[pl.ds(0, nch)])
                if with_gather:
                    pltpu.sync_copy(src_hbm.at[pl.ds(rowoff, nch)],
                                    src_v.at[pl.ds(0, nch)])

                @pl.loop(0, nch)
                def _(j):
                    if with_gather:
                        pltpu.async_copy(
                            val_hbm.at[src_v.at[j]], rows, sem).wait()
                    pltpu.sync_copy(rows, s_sh.at[dst_v.at[j]], add=True)

        plsc.subcore_barrier()
        for k in range(zr):
            base = (s * zr + k) * _CH
            pltpu.sync_copy(s_sh.at[pl.ds(base, _CH)], rows)
            pltpu.sync_copy(rows, out_ref.at[c, pl.ds(base, _CH)])

    if with_gather:
        def kern(val_hbm, src_hbm, dst_hbm, zeros_hbm, out_ref,
                 src_v, dst_v, rows, s_sh, sem):
            body((val_hbm, src_hbm, dst_hbm, zeros_hbm), out_ref,
                 (src_v, dst_v, rows, s_sh, sem))
    else:
        def kern(val_hbm, dst_hbm, zeros_hbm, out_ref, dst_v, rows, s_sh, sem):
            body((val_hbm, dst_hbm, zeros_hbm), out_ref,
                 (dst_v, rows, s_sh, sem))

    return pl.kernel(
        kern,
        out_type=jax.ShapeDtypeStruct((_NC, sh_rows, d), jnp.float32),
        mesh=_mesh(),
        scratch_types=scratch,
    )


def _t1_body(x_ref, wg_ref, deg_ref, hp_ref, dinv_ref):
    xb = x_ref[...]
    dg = deg_ref[...]
    degsum = dg[0, :, 0:1] + dg[1, :, 0:1] + 1.0
    dinv = lax.rsqrt(jnp.maximum(degsum, 1e-12))
    h = jnp.dot(xb, wg_ref[...], preferred_element_type=jnp.float32)
    dinvb = jnp.broadcast_to(dinv, xb.shape)
    hp_ref[...] = h * dinvb
    dinv_ref[...] = dinvb


def _t1_call(x, wg, deg, bt):
    n, d = x.shape
    grid = n // bt
    return pl.pallas_call(
        _t1_body,
        grid=(grid,),
        in_specs=[
            pl.BlockSpec((bt, d), lambda i: (i, 0)),
            pl.BlockSpec((d, d), lambda i: (0, 0)),
            pl.BlockSpec((_NC, bt, d), lambda i: (0, i, 0)),
        ],
        out_specs=[
            pl.BlockSpec((bt, d), lambda i: (i, 0)),
            pl.BlockSpec((bt, d), lambda i: (i, 0)),
        ],
        out_shape=[
            jax.ShapeDtypeStruct((n, d), jnp.float32),
            jax.ShapeDtypeStruct((n, d), jnp.float32),
        ],
    )(x, wg, deg)


def _ln(v, g, b):
    m = jnp.mean(v, axis=-1, keepdims=True)
    var = jnp.mean((v - m) ** 2, axis=-1, keepdims=True)
    return (v - m) * lax.rsqrt(var + _EPS) * g + b


def _t2_body(x_ref, hp_ref, dinv_ref, s_ref, bg_ref, g1_ref, b1_ref,
             w1_ref, c1_ref, w2_ref, c2_ref, g2_ref, b2_ref, out_ref):
    xb = x_ref[...]
    conv = dinv_ref[...] * (s_ref[0] + s_ref[1] + hp_ref[...]) + bg_ref[...]
    x1 = _ln(xb + conv, g1_ref[...], b1_ref[...])
    h = jnp.maximum(jnp.dot(x1, w1_ref[...], preferred_element_type=jnp.float32)
                    + c1_ref[...], 0.0)
    ffn = jnp.dot(h, w2_ref[...], preferred_element_type=jnp.float32) + c2_ref[...]
    out_ref[...] = _ln(x1 + ffn, g2_ref[...], b2_ref[...])


def _t2_call(x, hp, dinvb, s_part, p, bt):
    n, d = x.shape
    ff = p['W1'].shape[1]
    grid = n // bt
    row = lambda i: (i, 0)
    zero = lambda i: (0, 0)
    vec = lambda a: a.reshape(1, -1)
    return pl.pallas_call(
        _t2_body,
        grid=(grid,),
        in_specs=[
            pl.BlockSpec((bt, d), row),
            pl.BlockSpec((bt, d), row),
            pl.BlockSpec((bt, d), row),
            pl.BlockSpec((_NC, bt, d), lambda i: (0, i, 0)),
            pl.BlockSpec((1, d), zero),
            pl.BlockSpec((1, d), zero),
            pl.BlockSpec((1, d), zero),
            pl.BlockSpec((d, ff), zero),
            pl.BlockSpec((1, ff), zero),
            pl.BlockSpec((ff, d), zero),
            pl.BlockSpec((1, d), zero),
            pl.BlockSpec((1, d), zero),
            pl.BlockSpec((1, d), zero),
        ],
        out_specs=pl.BlockSpec((bt, d), row),
        out_shape=jax.ShapeDtypeStruct((n, d), jnp.float32),
    )(x, hp, dinvb, s_part, vec(p['bg']), vec(p['g1']), vec(p['b1']),
      p['W1'], vec(p['c1']), p['W2'], vec(p['c2']), vec(p['g2']), vec(p['b2']))


def kernel(x, edge_index, params):
    n, d = x.shape
    e = edge_index.shape[1]
    bt = 1000

    pair = -(-e // (_NS * _CH * 16)) * 16
    chunks0 = -(-int(pair * 0.65) // 8) * 8
    chunks1 = pair - chunks0
    cmax = max(chunks0, chunks1)
    rows_used = _NS * pair
    rows_pad = rows_used + cmax
    e_pad = rows_pad * _CH
    src = edge_index[0]
    dst = edge_index[1]
    pad = e_pad - e
    srcp = jnp.concatenate(
        [src, jnp.zeros((pad,), jnp.int32)]).reshape(rows_pad, _CH)
    dstp = jnp.concatenate(
        [dst, jnp.full((pad,), n, jnp.int32)]).reshape(rows_pad, _CH)

    zerosd = jnp.zeros((_CH, d), jnp.float32)
    onesd = jnp.ones((_CH, d), jnp.float32)

    deg = _make_scatter(n, d, chunks0, chunks1, with_gather=False)(
        onesd, dstp, zerosd)
    scat = _make_scatter(n, d, chunks0, chunks1, with_gather=True)

    for p in params:
        hp, dinvb = _t1_call(x, p['Wg'], deg, bt)
        s_part = scat(hp, srcp, dstp, zerosd)
        x = _t2_call(x, hp, dinvb, s_part, p, bt)
    return x


# speedup vs baseline: 2.5939x; 2.2167x over previous
import jax
import jax.numpy as jnp
from jax import lax
from jax.experimental import pallas as pl
from jax.experimental.pallas import tpu as pltpu
from jax.experimental.pallas import tpu_sc as plsc

_NC = 2
_NS = 16
_NW = _NC * _NS
_CH = 128
_EPS = 1e-5


def _mesh():
    return plsc.VectorSubcoreMesh(core_axis_name="c", subcore_axis_name="s")


def _acc_rows(n_nodes):
    return -(-(n_nodes + 1) // (_NS * _CH)) * (_NS * _CH)


def _make_scatter(n_nodes, d, chunks0, chunks1, with_gather):
    sh_rows = _acc_rows(n_nodes)
    zr = sh_rows // _NS // _CH
    cmax = max(chunks0, chunks1)

    scratch = [
        pltpu.VMEM((cmax, _CH), jnp.int32),
        pltpu.VMEM((_CH, d), jnp.float32),
        pltpu.VMEM_SHARED((sh_rows, d), jnp.float32),
        pltpu.SemaphoreType.DMA,
    ]
    if with_gather:
        scratch.insert(0, pltpu.VMEM((cmax, _CH), jnp.int32))

    def body(hbm_refs, out_ref, vmem_refs):
        if with_gather:
            val_hbm, src_hbm, dst_hbm, zeros_hbm = hbm_refs
            src_v, dst_v, rows, s_sh, sem = vmem_refs
        else:
            val_hbm, dst_hbm, zeros_hbm = hbm_refs
            dst_v, rows, s_sh, sem = vmem_refs
        c = lax.axis_index("c")
        s = lax.axis_index("s")
        pltpu.sync_copy(zeros_hbm, rows)
        for k in range(zr):
            pltpu.sync_copy(rows, s_sh.at[pl.ds((s * zr + k) * _CH, _CH)])
        plsc.subcore_barrier()

        if not with_gather:
            pltpu.sync_copy(val_hbm, rows)

        for cc, nch, base in ((0, chunks0, 0), (1, chunks1, _NS * chunks0)):
            @pl.when(c == cc)
            def _():
                rowoff = base + s * nch
                pltpu.sync_copy(dst_hbm.at[pl.ds(rowoff, nch)],
                                dst_v.at[pl.ds(0, nch)])
                if with_gather:
                    pltpu.sync_copy(src_hbm.at[pl.ds(rowoff, nch)],
                                    src_v.at[pl.ds(0, nch)])

                @pl.loop(0, nch)
                def _(j):
                    if with_gather:
                        pltpu.async_copy(
                            val_hbm.at[src_v.at[j]], rows, sem).wait()
                    pltpu.sync_copy(rows, s_sh.at[dst_v.at[j]], add=True)

        plsc.subcore_barrier()
        for k in range(zr):
            base = (s * zr + k) * _CH
            pltpu.sync_copy(s_sh.at[pl.ds(base, _CH)], rows)
            pltpu.sync_copy(rows, out_ref.at[c, pl.ds(base, _CH)])

    if with_gather:
        def kern(val_hbm, src_hbm, dst_hbm, zeros_hbm, out_ref,
                 src_v, dst_v, rows, s_sh, sem):
            body((val_hbm, src_hbm, dst_hbm, zeros_hbm), out_ref,
                 (src_v, dst_v, rows, s_sh, sem))
    else:
        def kern(val_hbm, dst_hbm, zeros_hbm, out_ref, dst_v, rows, s_sh, sem):
            body((val_hbm, dst_hbm, zeros_hbm), out_ref,
                 (dst_v, rows, s_sh, sem))

    return pl.kernel(
        kern,
        out_type=jax.ShapeDtypeStruct((_NC, sh_rows, d), jnp.float32),
        mesh=_mesh(),
        scratch_types=scratch,
    )


def _t1_body(x_ref, wg_ref, deg_ref, hp_ref, dinv_ref):
    xb = x_ref[...]
    dg = deg_ref[...]
    degsum = dg[0, :, 0:1] + dg[1, :, 0:1] + 1.0
    dinv = lax.rsqrt(jnp.maximum(degsum, 1e-12))
    h = jnp.dot(xb, wg_ref[...], preferred_element_type=jnp.float32)
    dinvb = jnp.broadcast_to(dinv, xb.shape)
    hp_ref[...] = h * dinvb
    dinv_ref[...] = dinvb


def _t1_call(x, wg, deg, bt):
    n, d = x.shape
    grid = n // bt
    return pl.pallas_call(
        _t1_body,
        grid=(grid,),
        in_specs=[
            pl.BlockSpec((bt, d), lambda i: (i, 0)),
            pl.BlockSpec((d, d), lambda i: (0, 0)),
            pl.BlockSpec((_NC, bt, d), lambda i: (0, i, 0)),
        ],
        out_specs=[
            pl.BlockSpec((bt, d), lambda i: (i, 0)),
            pl.BlockSpec((bt, d), lambda i: (i, 0)),
        ],
        out_shape=[
            jax.ShapeDtypeStruct((n, d), jnp.float32),
            jax.ShapeDtypeStruct((n, d), jnp.float32),
        ],
    )(x, wg, deg)


def _ln(v, g, b):
    m = jnp.mean(v, axis=-1, keepdims=True)
    var = jnp.mean((v - m) ** 2, axis=-1, keepdims=True)
    return (v - m) * lax.rsqrt(var + _EPS) * g + b


def _t2_body(x_ref, hp_ref, dinv_ref, s_ref, bg_ref, g1_ref, b1_ref,
             w1_ref, c1_ref, w2_ref, c2_ref, g2_ref, b2_ref, out_ref):
    xb = x_ref[...]
    conv = dinv_ref[...] * (s_ref[0] + s_ref[1] + hp_ref[...]) + bg_ref[...]
    x1 = _ln(xb + conv, g1_ref[...], b1_ref[...])
    h = jnp.maximum(jnp.dot(x1, w1_ref[...], preferred_element_type=jnp.float32)
                    + c1_ref[...], 0.0)
    ffn = jnp.dot(h, w2_ref[...], preferred_element_type=jnp.float32) + c2_ref[...]
    out_ref[...] = _ln(x1 + ffn, g2_ref[...], b2_ref[...])


def _t2_call(x, hp, dinvb, s_part, p, bt):
    n, d = x.shape
    ff = p['W1'].shape[1]
    grid = n // bt
    row = lambda i: (i, 0)
    zero = lambda i: (0, 0)
    vec = lambda a: a.reshape(1, -1)
    return pl.pallas_call(
        _t2_body,
        grid=(grid,),
        in_specs=[
            pl.BlockSpec((bt, d), row),
            pl.BlockSpec((bt, d), row),
            pl.BlockSpec((bt, d), row),
            pl.BlockSpec((_NC, bt, d), lambda i: (0, i, 0)),
            pl.BlockSpec((1, d), zero),
            pl.BlockSpec((1, d), zero),
            pl.BlockSpec((1, d), zero),
            pl.BlockSpec((d, ff), zero),
            pl.BlockSpec((1, ff), zero),
            pl.BlockSpec((ff, d), zero),
            pl.BlockSpec((1, d), zero),
            pl.BlockSpec((1, d), zero),
            pl.BlockSpec((1, d), zero),
        ],
        out_specs=pl.BlockSpec((bt, d), row),
        out_shape=jax.ShapeDtypeStruct((n, d), jnp.float32),
    )(x, hp, dinvb, s_part, vec(p['bg']), vec(p['g1']), vec(p['b1']),
      p['W1'], vec(p['c1']), p['W2'], vec(p['c2']), vec(p['g2']), vec(p['b2']))


def kernel(x, edge_index, params):
    n, d = x.shape
    e = edge_index.shape[1]
    bt = 1000

    pair = -(-e // (_NS * _CH * 16)) * 16
    chunks0 = pair // 2 // 8 * 8
    chunks1 = pair - chunks0
    rows_pad = _NS * pair
    e_pad = rows_pad * _CH
    src = edge_index[0]
    dst = edge_index[1]
    pad = e_pad - e
    spare = _acc_rows(n) - n
    fill = jnp.arange(pad, dtype=jnp.int32)
    srcp = jnp.concatenate(
        [src, fill % jnp.int32(n)]).reshape(rows_pad, _CH)
    dstp = jnp.concatenate(
        [dst, jnp.int32(n) + fill % jnp.int32(spare)]).reshape(rows_pad, _CH)

    zerosd = jnp.zeros((_CH, d), jnp.float32)
    onesd = jnp.ones((_CH, d), jnp.float32)

    deg = _make_scatter(n, d, chunks0, chunks1, with_gather=False)(
        onesd, dstp, zerosd)
    scat = _make_scatter(n, d, chunks0, chunks1, with_gather=True)

    for p in params:
        hp, dinvb = _t1_call(x, p['Wg'], deg, bt)
        s_part = scat(hp, srcp, dstp, zerosd)
        x = _t2_call(x, hp, dinvb, s_part, p, bt)
    return x


# speedup vs baseline: 3.1720x; 1.2229x over previous
import jax
import jax.numpy as jnp
from jax import lax
from jax.experimental import pallas as pl
from jax.experimental.pallas import tpu as pltpu
from jax.experimental.pallas import tpu_sc as plsc

_NC = 2
_NS = 16
_NW = _NC * _NS
_CH = 128
_EPS = 1e-5


def _mesh():
    return plsc.VectorSubcoreMesh(core_axis_name="c", subcore_axis_name="s")


def _acc_rows(n_nodes):
    return -(-(n_nodes + 1) // (_NS * _CH)) * (_NS * _CH)


def _make_scatter(n_nodes, d, phases, chunks, with_gather):
    sh_rows = _acc_rows(n_nodes)
    zr = sh_rows // _NS // _CH
    assert chunks % 2 == 0

    scratch = [
        pltpu.VMEM((chunks, _CH), jnp.int32),
        pltpu.VMEM((_CH, d), jnp.float32),
        pltpu.VMEM_SHARED((sh_rows, d), jnp.float32),
        pltpu.SemaphoreType.DMA,
    ]
    if with_gather:
        scratch.insert(0, pltpu.VMEM((chunks, _CH), jnp.int32))
        scratch.insert(3, pltpu.VMEM((_CH, d), jnp.float32))
        scratch.append(pltpu.SemaphoreType.DMA)

    def body(val_hbm, src_hbm, dst_hbm, zeros_hbm, out_ref,
             src_v, dst_v, rows0, rows1, s_sh, sem0, sem1):
        c = lax.axis_index("c")
        s = lax.axis_index("s")
        tile_row = (c * _NS + s) * phases * chunks
        pltpu.sync_copy(zeros_hbm, rows0)
        for k in range(zr):
            pltpu.sync_copy(rows0, s_sh.at[pl.ds((s * zr + k) * _CH, _CH)])
        plsc.subcore_barrier()

        if not with_gather:
            pltpu.sync_copy(val_hbm, rows0)
            for ph in range(phases):
                pltpu.sync_copy(
                    dst_hbm.at[pl.ds(tile_row + ph * chunks, chunks)], dst_v)

                @pl.loop(0, chunks)
                def _(j):
                    pltpu.sync_copy(rows0, s_sh.at[dst_v.at[j]], add=True)
        else:
            bufs = ((rows0, sem0), (rows1, sem1))
            for ph in range(phases):
                pltpu.sync_copy(
                    dst_hbm.at[pl.ds(tile_row + ph * chunks, chunks)], dst_v)
                pltpu.sync_copy(
                    src_hbm.at[pl.ds(tile_row + ph * chunks, chunks)], src_v)
                pltpu.async_copy(val_hbm.at[src_v.at[0]], rows0, sem0)

                @pl.loop(0, chunks, step=2)
                def _(j):
                    for b in range(2):
                        jj = j + b
                        rb, sb = bufs[b]
                        rn, sn = bufs[1 - b]
                        nxt = jnp.where(jj + 1 == chunks, 0, jj + 1)
                        pltpu.make_async_copy(
                            val_hbm.at[src_v.at[jj]], rb, sb).wait()
                        pltpu.async_copy(val_hbm.at[src_v.at[nxt]], rn, sn)
                        pltpu.sync_copy(rb, s_sh.at[dst_v.at[jj]], add=True)

                pltpu.make_async_copy(
                    val_hbm.at[src_v.at[0]], rows0, sem0).wait()

        plsc.subcore_barrier()
        for k in range(zr):
            base = (s * zr + k) * _CH
            pltpu.sync_copy(s_sh.at[pl.ds(base, _CH)], rows0)
            pltpu.sync_copy(rows0, out_ref.at[c, pl.ds(base, _CH)])

    if with_gather:
        def kern(val_hbm, src_hbm, dst_hbm, zeros_hbm, out_ref,
                 src_v, dst_v, rows0, rows1, s_sh, sem0, sem1):
            body(val_hbm, src_hbm, dst_hbm, zeros_hbm, out_ref,
                 src_v, dst_v, rows0, rows1, s_sh, sem0, sem1)
    else:
        def kern(val_hbm, dst_hbm, zeros_hbm, out_ref, dst_v, rows, s_sh, sem):
            body(val_hbm, None, dst_hbm, zeros_hbm, out_ref,
                 None, dst_v, rows, None, s_sh, sem, None)

    return pl.kernel(
        kern,
        out_type=jax.ShapeDtypeStruct((_NC, sh_rows, d), jnp.float32),
        mesh=_mesh(),
        scratch_types=scratch,
    )


def _t1_body(x_ref, wg_ref, deg_ref, hp_ref, dinv_ref):
    xb = x_ref[...]
    dg = deg_ref[...]
    degsum = dg[0, :, 0:1] + dg[1, :, 0:1] + 1.0
    dinv = lax.rsqrt(jnp.maximum(degsum, 1e-12))
    h = jnp.dot(xb, wg_ref[...], preferred_element_type=jnp.float32)
    dinvb = jnp.broadcast_to(dinv, xb.shape)
    hp_ref[...] = h * dinvb
    dinv_ref[...] = dinvb


def _t1_call(x, wg, deg, bt):
    n, d = x.shape
    grid = n // bt
    return pl.pallas_call(
        _t1_body,
        grid=(grid,),
        in_specs=[
            pl.BlockSpec((bt, d), lambda i: (i, 0)),
            pl.BlockSpec((d, d), lambda i: (0, 0)),
            pl.BlockSpec((_NC, bt, d), lambda i: (0, i, 0)),
        ],
        out_specs=[
            pl.BlockSpec((bt, d), lambda i: (i, 0)),
            pl.BlockSpec((bt, d), lambda i: (i, 0)),
        ],
        out_shape=[
            jax.ShapeDtypeStruct((n, d), jnp.float32),
            jax.ShapeDtypeStruct((n, d), jnp.float32),
        ],
    )(x, wg, deg)


def _ln(v, g, b):
    m = jnp.mean(v, axis=-1, keepdims=True)
    var = jnp.mean((v - m) ** 2, axis=-1, keepdims=True)
    return (v - m) * lax.rsqrt(var + _EPS) * g + b


def _t2_body(x_ref, hp_ref, dinv_ref, s_ref, bg_ref, g1_ref, b1_ref,
             w1_ref, c1_ref, w2_ref, c2_ref, g2_ref, b2_ref, out_ref):
    xb = x_ref[...]
    conv = dinv_ref[...] * (s_ref[0] + s_ref[1] + hp_ref[...]) + bg_ref[...]
    x1 = _ln(xb + conv, g1_ref[...], b1_ref[...])
    h = jnp.maximum(jnp.dot(x1, w1_ref[...], preferred_element_type=jnp.float32)
                    + c1_ref[...], 0.0)
    ffn = jnp.dot(h, w2_ref[...], preferred_element_type=jnp.float32) + c2_ref[...]
    out_ref[...] = _ln(x1 + ffn, g2_ref[...], b2_ref[...])


def _t2_call(x, hp, dinvb, s_part, p, bt):
    n, d = x.shape
    ff = p['W1'].shape[1]
    grid = n // bt
    row = lambda i: (i, 0)
    zero = lambda i: (0, 0)
    vec = lambda a: a.reshape(1, -1)
    return pl.pallas_call(
        _t2_body,
        grid=(grid,),
        in_specs=[
            pl.BlockSpec((bt, d), row),
            pl.BlockSpec((bt, d), row),
            pl.BlockSpec((bt, d), row),
            pl.BlockSpec((_NC, bt, d), lambda i: (0, i, 0)),
            pl.BlockSpec((1, d), zero),
            pl.BlockSpec((1, d), zero),
            pl.BlockSpec((1, d), zero),
            pl.BlockSpec((d, ff), zero),
            pl.BlockSpec((1, ff), zero),
            pl.BlockSpec((ff, d), zero),
            pl.BlockSpec((1, d), zero),
            pl.BlockSpec((1, d), zero),
            pl.BlockSpec((1, d), zero),
        ],
        out_specs=pl.BlockSpec((bt, d), row),
        out_shape=jax.ShapeDtypeStruct((n, d), jnp.float32),
    )(x, hp, dinvb, s_part, vec(p['bg']), vec(p['g1']), vec(p['b1']),
      p['W1'], vec(p['c1']), p['W2'], vec(p['c2']), vec(p['g2']), vec(p['b2']))


def kernel(x, edge_index, params):
    n, d = x.shape
    e = edge_index.shape[1]
    bt = 1000

    phases = 2
    chunks = -(-e // (_NW * _CH * phases * 4)) * 4
    rows_pad = _NW * phases * chunks
    e_pad = rows_pad * _CH
    src = edge_index[0]
    dst = edge_index[1]
    pad = e_pad - e
    spare = _acc_rows(n) - n
    fill = jnp.arange(pad, dtype=jnp.int32)
    srcp = jnp.concatenate(
        [src, fill % jnp.int32(n)]).reshape(rows_pad, _CH)
    dstp = jnp.concatenate(
        [dst, jnp.int32(n) + fill % jnp.int32(spare)]).reshape(rows_pad, _CH)

    zerosd = jnp.zeros((_CH, d), jnp.float32)
    onesd = jnp.ones((_CH, d), jnp.float32)

    deg = _make_scatter(n, d, phases, chunks, with_gather=False)(
        onesd, dstp, zerosd)
    scat = _make_scatter(n, d, phases, chunks, with_gather=True)

    for p in params:
        hp, dinvb = _t1_call(x, p['Wg'], deg, bt)
        s_part = scat(hp, srcp, dstp, zerosd)
        x = _t2_call(x, hp, dinvb, s_part, p, bt)
    return x


# speedup vs baseline: 3.3001x; 1.0404x over previous
import jax
import jax.numpy as jnp
from jax import lax
from jax.experimental import pallas as pl
from jax.experimental.pallas import tpu as pltpu
from jax.experimental.pallas import tpu_sc as plsc

_NC = 2
_NS = 16
_NW = _NC * _NS
_CH = 128
_EPS = 1e-5


def _mesh():
    return plsc.VectorSubcoreMesh(core_axis_name="c", subcore_axis_name="s")


def _acc_rows(n_nodes):
    return -(-(n_nodes + 1) // (_NS * _CH)) * (_NS * _CH)


def _make_scatter(n_nodes, d, phases, chunks, with_gather):
    sh_rows = _acc_rows(n_nodes)
    zr = sh_rows // _NS // _CH
    assert chunks % 2 == 0

    scratch = [
        pltpu.VMEM((chunks, _CH), jnp.int32),
        pltpu.VMEM((_CH, d), jnp.float32),
        pltpu.VMEM_SHARED((sh_rows, d), jnp.float32),
        pltpu.SemaphoreType.DMA,
    ]
    if with_gather:
        scratch.insert(0, pltpu.VMEM((chunks, _CH), jnp.int32))
        scratch.insert(3, pltpu.VMEM((_CH, d), jnp.float32))
        scratch.append(pltpu.SemaphoreType.DMA)

    def body(val_hbm, src_hbm, dst_hbm, zeros_hbm, out_ref,
             src_v, dst_v, rows0, rows1, s_sh, sem0, sem1):
        c = lax.axis_index("c")
        s = lax.axis_index("s")
        tile_row = (c * _NS + s) * phases * chunks
        pltpu.sync_copy(zeros_hbm, rows0)
        for k in range(zr):
            pltpu.sync_copy(rows0, s_sh.at[pl.ds((s * zr + k) * _CH, _CH)])
        plsc.subcore_barrier()

        if not with_gather:
            pltpu.sync_copy(val_hbm, rows0)
            for ph in range(phases):
                pltpu.sync_copy(
                    dst_hbm.at[pl.ds(tile_row + ph * chunks, chunks)], dst_v)

                @pl.loop(0, chunks)
                def _(j):
                    pltpu.sync_copy(rows0, s_sh.at[dst_v.at[j]], add=True)
        else:
            bufs = ((rows0, sem0), (rows1, sem1))
            for ph in range(phases):
                pltpu.sync_copy(
                    dst_hbm.at[pl.ds(tile_row + ph * chunks, chunks)], dst_v)
                pltpu.sync_copy(
                    src_hbm.at[pl.ds(tile_row + ph * chunks, chunks)], src_v)
                pltpu.async_copy(val_hbm.at[src_v.at[0]], rows0, sem0)

                @pl.loop(0, chunks, step=2)
                def _(j):
                    for b in range(2):
                        jj = j + b
                        rb, sb = bufs[b]
                        rn, sn = bufs[1 - b]
                        nxt = jnp.where(jj + 1 == chunks, 0, jj + 1)
                        pltpu.make_async_copy(
                            val_hbm.at[src_v.at[jj]], rb, sb).wait()
                        pltpu.async_copy(val_hbm.at[src_v.at[nxt]], rn, sn)
                        pltpu.sync_copy(rb, s_sh.at[dst_v.at[jj]], add=True)

                pltpu.make_async_copy(
                    val_hbm.at[src_v.at[0]], rows0, sem0).wait()

        plsc.subcore_barrier()
        for k in range(zr):
            base = (s * zr + k) * _CH
            pltpu.sync_copy(s_sh.at[pl.ds(base, _CH)], rows0)
            pltpu.sync_copy(rows0, out_ref.at[c, pl.ds(base, _CH)])

    if with_gather:
        def kern(val_hbm, src_hbm, dst_hbm, zeros_hbm, out_ref,
                 src_v, dst_v, rows0, rows1, s_sh, sem0, sem1):
            body(val_hbm, src_hbm, dst_hbm, zeros_hbm, out_ref,
                 src_v, dst_v, rows0, rows1, s_sh, sem0, sem1)
    else:
        def kern(val_hbm, dst_hbm, zeros_hbm, out_ref, dst_v, rows, s_sh, sem):
            body(val_hbm, None, dst_hbm, zeros_hbm, out_ref,
                 None, dst_v, rows, None, s_sh, sem, None)

    return pl.kernel(
        kern,
        out_type=jax.ShapeDtypeStruct((_NC, sh_rows, d), jnp.float32),
        mesh=_mesh(),
        scratch_types=scratch,
    )


def _t1_body(x_ref, wg_ref, deg_ref, hp_ref, dinv_ref):
    xb = x_ref[...]
    dg = deg_ref[...]
    degsum = dg[0, :, 0:1] + dg[1, :, 0:1] + 1.0
    dinv = lax.rsqrt(jnp.maximum(degsum, 1e-12))
    h = jnp.dot(xb, wg_ref[...], preferred_element_type=jnp.float32)
    dinvb = jnp.broadcast_to(dinv, xb.shape)
    hp_ref[...] = h * dinvb
    dinv_ref[...] = dinvb


def _t1_call(x, wg, deg, bt):
    n, d = x.shape
    grid = n // bt
    return pl.pallas_call(
        _t1_body,
        grid=(grid,),
        in_specs=[
            pl.BlockSpec((bt, d), lambda i: (i, 0)),
            pl.BlockSpec((d, d), lambda i: (0, 0)),
            pl.BlockSpec((_NC, bt, d), lambda i: (0, i, 0)),
        ],
        out_specs=[
            pl.BlockSpec((bt, d), lambda i: (i, 0)),
            pl.BlockSpec((bt, d), lambda i: (i, 0)),
        ],
        out_shape=[
            jax.ShapeDtypeStruct((n, d), jnp.float32),
            jax.ShapeDtypeStruct((n, d), jnp.float32),
        ],
    )(x, wg, deg)


def _ln(v, g, b):
    m = jnp.mean(v, axis=-1, keepdims=True)
    var = jnp.mean((v - m) ** 2, axis=-1, keepdims=True)
    return (v - m) * lax.rsqrt(var + _EPS) * g + b


def _t2_body(x_ref, hp_ref, dinv_ref, s_ref, bg_ref, g1_ref, b1_ref,
             w1_ref, c1_ref, w2_ref, c2_ref, g2_ref, b2_ref, out_ref):
    xb = x_ref[...]
    conv = dinv_ref[...] * (s_ref[0] + s_ref[1] + hp_ref[...]) + bg_ref[...]
    x1 = _ln(xb + conv, g1_ref[...], b1_ref[...])
    h = jnp.maximum(jnp.dot(x1, w1_ref[...], preferred_element_type=jnp.float32)
                    + c1_ref[...], 0.0)
    ffn = jnp.dot(h, w2_ref[...], preferred_element_type=jnp.float32) + c2_ref[...]
    out_ref[...] = _ln(x1 + ffn, g2_ref[...], b2_ref[...])


def _t2_call(x, hp, dinvb, s_part, p, bt):
    n, d = x.shape
    ff = p['W1'].shape[1]
    grid = n // bt
    row = lambda i: (i, 0)
    zero = lambda i: (0, 0)
    vec = lambda a: a.reshape(1, -1)
    return pl.pallas_call(
        _t2_body,
        grid=(grid,),
        in_specs=[
            pl.BlockSpec((bt, d), row),
            pl.BlockSpec((bt, d), row),
            pl.BlockSpec((bt, d), row),
            pl.BlockSpec((_NC, bt, d), lambda i: (0, i, 0)),
            pl.BlockSpec((1, d), zero),
            pl.BlockSpec((1, d), zero),
            pl.BlockSpec((1, d), zero),
            pl.BlockSpec((d, ff), zero),
            pl.BlockSpec((1, ff), zero),
            pl.BlockSpec((ff, d), zero),
            pl.BlockSpec((1, d), zero),
            pl.BlockSpec((1, d), zero),
            pl.BlockSpec((1, d), zero),
        ],
        out_specs=pl.BlockSpec((bt, d), row),
        out_shape=jax.ShapeDtypeStruct((n, d), jnp.float32),
    )(x, hp, dinvb, s_part, vec(p['bg']), vec(p['g1']), vec(p['b1']),
      p['W1'], vec(p['c1']), p['W2'], vec(p['c2']), vec(p['g2']), vec(p['b2']))


def _t21_body(x_ref, hp_ref, dinv_ref, s_ref, bg_ref, g1_ref, b1_ref,
              w1_ref, c1_ref, w2_ref, c2_ref, g2_ref, b2_ref, wg_ref,
              out_ref, hp2_ref):
    xb = x_ref[...]
    conv = dinv_ref[...] * (s_ref[0] + s_ref[1] + hp_ref[...]) + bg_ref[...]
    x1 = _ln(xb + conv, g1_ref[...], b1_ref[...])
    h = jnp.maximum(jnp.dot(x1, w1_ref[...], preferred_element_type=jnp.float32)
                    + c1_ref[...], 0.0)
    ffn = jnp.dot(h, w2_ref[...], preferred_element_type=jnp.float32) + c2_ref[...]
    x2 = _ln(x1 + ffn, g2_ref[...], b2_ref[...])
    out_ref[...] = x2
    hp2_ref[...] = jnp.dot(
        x2, wg_ref[...], preferred_element_type=jnp.float32) * dinv_ref[...]


def _t21_call(x, hp, dinvb, s_part, p, wg2, bt):
    n, d = x.shape
    ff = p['W1'].shape[1]
    grid = n // bt
    row = lambda i: (i, 0)
    zero = lambda i: (0, 0)
    vec = lambda a: a.reshape(1, -1)
    return pl.pallas_call(
        _t21_body,
        grid=(grid,),
        in_specs=[
            pl.BlockSpec((bt, d), row),
            pl.BlockSpec((bt, d), row),
            pl.BlockSpec((bt, d), row),
            pl.BlockSpec((_NC, bt, d), lambda i: (0, i, 0)),
            pl.BlockSpec((1, d), zero),
            pl.BlockSpec((1, d), zero),
            pl.BlockSpec((1, d), zero),
            pl.BlockSpec((d, ff), zero),
            pl.BlockSpec((1, ff), zero),
            pl.BlockSpec((ff, d), zero),
            pl.BlockSpec((1, d), zero),
            pl.BlockSpec((1, d), zero),
            pl.BlockSpec((1, d), zero),
            pl.BlockSpec((d, d), zero),
        ],
        out_specs=[pl.BlockSpec((bt, d), row), pl.BlockSpec((bt, d), row)],
        out_shape=[
            jax.ShapeDtypeStruct((n, d), jnp.float32),
            jax.ShapeDtypeStruct((n, d), jnp.float32),
        ],
    )(x, hp, dinvb, s_part, vec(p['bg']), vec(p['g1']), vec(p['b1']),
      p['W1'], vec(p['c1']), p['W2'], vec(p['c2']), vec(p['g2']),
      vec(p['b2']), wg2)


def kernel(x, edge_index, params):
    n, d = x.shape
    e = edge_index.shape[1]
    bt = 1000

    phases = 2
    chunks = -(-e // (_NW * _CH * phases * 4)) * 4
    rows_pad = _NW * phases * chunks
    e_pad = rows_pad * _CH
    src = edge_index[0]
    dst = edge_index[1]
    pad = e_pad - e
    spare = _acc_rows(n) - n
    fill = jnp.arange(pad, dtype=jnp.int32)
    srcp = jnp.concatenate(
        [src, fill % jnp.int32(n)]).reshape(rows_pad, _CH)
    dstp = jnp.concatenate(
        [dst, jnp.int32(n) + fill % jnp.int32(spare)]).reshape(rows_pad, _CH)

    zerosd = jnp.zeros((_CH, d), jnp.float32)
    onesd = jnp.ones((_CH, d), jnp.float32)

    deg = _make_scatter(n, d, phases, chunks, with_gather=False)(
        onesd, dstp, zerosd)
    scat = _make_scatter(n, d, phases, chunks, with_gather=True)

    hp, dinvb = _t1_call(x, params[0]['Wg'], deg, bt)
    for l, p in enumerate(params):
        s_part = scat(hp, srcp, dstp, zerosd)
        if l + 1 < len(params):
            x, hp = _t21_call(x, hp, dinvb, s_part, p, params[l + 1]['Wg'], bt)
        else:
            x = _t2_call(x, hp, dinvb, s_part, p, bt)
    return x


# speedup vs baseline: 3.3393x; 1.0119x over previous
import jax
import jax.numpy as jnp
from jax import lax
from jax.experimental import pallas as pl
from jax.experimental.pallas import tpu as pltpu
from jax.experimental.pallas import tpu_sc as plsc

_NC = 2
_NS = 16
_NW = _NC * _NS
_CH = 128
_EPS = 1e-5


def _mesh():
    return plsc.VectorSubcoreMesh(core_axis_name="c", subcore_axis_name="s")


def _acc_rows(n_nodes):
    return -(-(n_nodes + 1) // (_NS * _CH)) * (_NS * _CH)


def _make_scatter(n_nodes, d, phases, chunks, with_gather):
    sh_rows = _acc_rows(n_nodes)
    zr = sh_rows // _NS // _CH
    assert chunks % 2 == 0

    scratch = [
        pltpu.VMEM((chunks, _CH), jnp.int32),
        pltpu.VMEM((_CH, d), jnp.float32),
        pltpu.VMEM_SHARED((sh_rows, d), jnp.float32),
        pltpu.SemaphoreType.DMA,
    ]
    if with_gather:
        scratch.insert(0, pltpu.VMEM((chunks, _CH), jnp.int32))
        scratch.insert(3, pltpu.VMEM((_CH, d), jnp.float32))
        scratch.append(pltpu.SemaphoreType.DMA)

    def body(val_hbm, src_hbm, dst_hbm, zeros_hbm, out_ref,
             src_v, dst_v, rows0, rows1, s_sh, sem0, sem1):
        c = lax.axis_index("c")
        s = lax.axis_index("s")
        tile_row = (c * _NS + s) * phases * chunks
        pltpu.sync_copy(zeros_hbm, rows0)
        for k in range(zr):
            pltpu.sync_copy(rows0, s_sh.at[pl.ds((s * zr + k) * _CH, _CH)])
        plsc.subcore_barrier()

        if not with_gather:
            pltpu.sync_copy(val_hbm, rows0)
            for ph in range(phases):
                pltpu.sync_copy(
                    dst_hbm.at[pl.ds(tile_row + ph * chunks, chunks)], dst_v)

                @pl.loop(0, chunks)
                def _(j):
                    pltpu.sync_copy(rows0, s_sh.at[dst_v.at[j]], add=True)
        else:
            bufs = ((rows0, sem0), (rows1, sem1))
            for ph in range(phases):
                pltpu.sync_copy(
                    dst_hbm.at[pl.ds(tile_row + ph * chunks, chunks)], dst_v)
                pltpu.sync_copy(
                    src_hbm.at[pl.ds(tile_row + ph * chunks, chunks)], src_v)
                pltpu.async_copy(val_hbm.at[src_v.at[0]], rows0, sem0)

                @pl.loop(0, chunks, step=2)
                def _(j):
                    for b in range(2):
                        jj = j + b
                        rb, sb = bufs[b]
                        rn, sn = bufs[1 - b]
                        nxt = jnp.where(jj + 1 == chunks, 0, jj + 1)
                        pltpu.make_async_copy(
                            val_hbm.at[src_v.at[jj]], rb, sb).wait()
                        pltpu.async_copy(val_hbm.at[src_v.at[nxt]], rn, sn)
                        pltpu.sync_copy(rb, s_sh.at[dst_v.at[jj]], add=True)

                pltpu.make_async_copy(
                    val_hbm.at[src_v.at[0]], rows0, sem0).wait()

        plsc.subcore_barrier()
        for k in range(zr):
            base = (s * zr + k) * _CH
            pltpu.sync_copy(s_sh.at[pl.ds(base, _CH)], rows0)
            pltpu.sync_copy(rows0, out_ref.at[c, pl.ds(base, _CH)])

    if with_gather:
        def kern(val_hbm, src_hbm, dst_hbm, zeros_hbm, out_ref,
                 src_v, dst_v, rows0, rows1, s_sh, sem0, sem1):
            body(val_hbm, src_hbm, dst_hbm, zeros_hbm, out_ref,
                 src_v, dst_v, rows0, rows1, s_sh, sem0, sem1)
    else:
        def kern(val_hbm, dst_hbm, zeros_hbm, out_ref, dst_v, rows, s_sh, sem):
            body(val_hbm, None, dst_hbm, zeros_hbm, out_ref,
                 None, dst_v, rows, None, s_sh, sem, None)

    return pl.kernel(
        kern,
        out_type=jax.ShapeDtypeStruct((_NC, sh_rows, d), jnp.float32),
        mesh=_mesh(),
        scratch_types=scratch,
    )


def _t1_body(x_ref, wg_ref, deg_ref, hp_ref, dinv_ref):
    xb = x_ref[...]
    dg = deg_ref[...]
    degsum = dg[0, :, 0:1] + dg[1, :, 0:1] + 1.0
    dinv = lax.rsqrt(jnp.maximum(degsum, 1e-12))
    h = jnp.dot(xb, wg_ref[...], preferred_element_type=jnp.float32)
    dinvb = jnp.broadcast_to(dinv, xb.shape)
    hp_ref[...] = h * dinvb
    dinv_ref[...] = dinvb


def _t1_call(x, wg, deg, bt):
    n, d = x.shape
    grid = n // bt
    return pl.pallas_call(
        _t1_body,
        grid=(grid,),
        in_specs=[
            pl.BlockSpec((bt, d), lambda i: (i, 0)),
            pl.BlockSpec((d, d), lambda i: (0, 0)),
            pl.BlockSpec((_NC, bt, d), lambda i: (0, i, 0)),
        ],
        out_specs=[
            pl.BlockSpec((bt, d), lambda i: (i, 0)),
            pl.BlockSpec((bt, d), lambda i: (i, 0)),
        ],
        out_shape=[
            jax.ShapeDtypeStruct((n, d), jnp.float32),
            jax.ShapeDtypeStruct((n, d), jnp.float32),
        ],
    )(x, wg, deg)


def _ln(v, g, b):
    m = jnp.mean(v, axis=-1, keepdims=True)
    var = jnp.mean((v - m) ** 2, axis=-1, keepdims=True)
    return (v - m) * lax.rsqrt(var + _EPS) * g + b


def _t2_body(x_ref, hp_ref, dinv_ref, s_ref, bg_ref, g1_ref, b1_ref,
             w1_ref, c1_ref, w2_ref, c2_ref, g2_ref, b2_ref, out_ref):
    xb = x_ref[...]
    conv = dinv_ref[...] * (s_ref[0] + s_ref[1] + hp_ref[...]) + bg_ref[...]
    x1 = _ln(xb + conv, g1_ref[...], b1_ref[...])
    h = jnp.maximum(jnp.dot(x1, w1_ref[...], preferred_element_type=jnp.float32)
                    + c1_ref[...], 0.0)
    ffn = jnp.dot(h, w2_ref[...], preferred_element_type=jnp.float32) + c2_ref[...]
    out_ref[...] = _ln(x1 + ffn, g2_ref[...], b2_ref[...])


def _t2_call(x, hp, dinvb, s_part, p, bt):
    n, d = x.shape
    ff = p['W1'].shape[1]
    grid = n // bt
    row = lambda i: (i, 0)
    zero = lambda i: (0, 0)
    vec = lambda a: a.reshape(1, -1)
    return pl.pallas_call(
        _t2_body,
        grid=(grid,),
        in_specs=[
            pl.BlockSpec((bt, d), row),
            pl.BlockSpec((bt, d), row),
            pl.BlockSpec((bt, d), row),
            pl.BlockSpec((_NC, bt, d), lambda i: (0, i, 0)),
            pl.BlockSpec((1, d), zero),
            pl.BlockSpec((1, d), zero),
            pl.BlockSpec((1, d), zero),
            pl.BlockSpec((d, ff), zero),
            pl.BlockSpec((1, ff), zero),
            pl.BlockSpec((ff, d), zero),
            pl.BlockSpec((1, d), zero),
            pl.BlockSpec((1, d), zero),
            pl.BlockSpec((1, d), zero),
        ],
        out_specs=pl.BlockSpec((bt, d), row),
        out_shape=jax.ShapeDtypeStruct((n, d), jnp.float32),
    )(x, hp, dinvb, s_part, vec(p['bg']), vec(p['g1']), vec(p['b1']),
      p['W1'], vec(p['c1']), p['W2'], vec(p['c2']), vec(p['g2']), vec(p['b2']))


def _t21_body(x_ref, hp_ref, dinv_ref, s_ref, bg_ref, g1_ref, b1_ref,
              w1_ref, c1_ref, w2_ref, c2_ref, g2_ref, b2_ref, wg_ref,
              out_ref, hp2_ref):
    xb = x_ref[...]
    conv = dinv_ref[...] * (s_ref[0] + s_ref[1] + hp_ref[...]) + bg_ref[...]
    x1 = _ln(xb + conv, g1_ref[...], b1_ref[...])
    h = jnp.maximum(jnp.dot(x1, w1_ref[...], preferred_element_type=jnp.float32)
                    + c1_ref[...], 0.0)
    ffn = jnp.dot(h, w2_ref[...], preferred_element_type=jnp.float32) + c2_ref[...]
    x2 = _ln(x1 + ffn, g2_ref[...], b2_ref[...])
    out_ref[...] = x2
    hp2_ref[...] = jnp.dot(
        x2, wg_ref[...], preferred_element_type=jnp.float32) * dinv_ref[...]


def _t21_call(x, hp, dinvb, s_part, p, wg2, bt):
    n, d = x.shape
    ff = p['W1'].shape[1]
    grid = n // bt
    row = lambda i: (i, 0)
    zero = lambda i: (0, 0)
    vec = lambda a: a.reshape(1, -1)
    return pl.pallas_call(
        _t21_body,
        grid=(grid,),
        in_specs=[
            pl.BlockSpec((bt, d), row),
            pl.BlockSpec((bt, d), row),
            pl.BlockSpec((bt, d), row),
            pl.BlockSpec((_NC, bt, d), lambda i: (0, i, 0)),
            pl.BlockSpec((1, d), zero),
            pl.BlockSpec((1, d), zero),
            pl.BlockSpec((1, d), zero),
            pl.BlockSpec((d, ff), zero),
            pl.BlockSpec((1, ff), zero),
            pl.BlockSpec((ff, d), zero),
            pl.BlockSpec((1, d), zero),
            pl.BlockSpec((1, d), zero),
            pl.BlockSpec((1, d), zero),
            pl.BlockSpec((d, d), zero),
        ],
        out_specs=[pl.BlockSpec((bt, d), row), pl.BlockSpec((bt, d), row)],
        out_shape=[
            jax.ShapeDtypeStruct((n, d), jnp.float32),
            jax.ShapeDtypeStruct((n, d), jnp.float32),
        ],
    )(x, hp, dinvb, s_part, vec(p['bg']), vec(p['g1']), vec(p['b1']),
      p['W1'], vec(p['c1']), p['W2'], vec(p['c2']), vec(p['g2']),
      vec(p['b2']), wg2)


def kernel(x, edge_index, params):
    n, d = x.shape
    e = edge_index.shape[1]
    bt = 2000

    phases = 2
    chunks = -(-e // (_NW * _CH * phases * 4)) * 4
    rows_pad = _NW * phases * chunks
    e_pad = rows_pad * _CH
    src = edge_index[0]
    dst = edge_index[1]
    pad = e_pad - e
    spare = _acc_rows(n) - n
    fill = jnp.arange(pad, dtype=jnp.int32)
    srcp = jnp.concatenate(
        [src, fill % jnp.int32(n)]).reshape(rows_pad, _CH)
    dstp = jnp.concatenate(
        [dst, jnp.int32(n) + fill % jnp.int32(spare)]).reshape(rows_pad, _CH)

    zerosd = jnp.zeros((_CH, d), jnp.float32)
    onesd = jnp.ones((_CH, d), jnp.float32)

    deg = _make_scatter(n, d, phases, chunks, with_gather=False)(
        onesd, dstp, zerosd)
    scat = _make_scatter(n, d, phases, chunks, with_gather=True)

    hp, dinvb = _t1_call(x, params[0]['Wg'], deg, bt)
    for l, p in enumerate(params):
        s_part = scat(hp, srcp, dstp, zerosd)
        if l + 1 < len(params):
            x, hp = _t21_call(x, hp, dinvb, s_part, p, params[l + 1]['Wg'], bt)
        else:
            x = _t2_call(x, hp, dinvb, s_part, p, bt)
    return x
